# Initial kernel scaffold; baseline (speedup 1.0000x reference)
import jax
import jax.numpy as jnp
from jax.experimental import pallas as pl


def kernel(features, edge_index, edge_weight, kernel, bias, skip_weight):
    raise NotImplementedError("write your pallas kernel here")


# speedup vs baseline: 4.4300x; 4.4300x over previous
import functools

import jax
import jax.numpy as jnp
from jax import lax
from jax.experimental import pallas as pl
from jax.experimental.pallas import tpu as pltpu
from jax.experimental.pallas import tpu_sc as plsc

_SELU_SCALE = 1.0507009873554805
_SELU_ALPHA = 1.6732632423543772


@functools.lru_cache(maxsize=None)
def _make_sc_scatter(N, D, E):
    NC, NS = 2, 16
    NW = NC * NS
    EW = E // NW
    K = 80
    NCH = EW // K
    assert EW % K == 0 and D % 16 == 0
    ROWS_PER_TILE = (N // NS) // 8 * 8
    REM_ROWS = N - NS * ROWS_PER_TILE
    ZROWS = 104
    assert ROWS_PER_TILE % ZROWS == 0 and REM_ROWS % 8 == 0
    assert REM_ROWS <= ZROWS

    mesh = plsc.VectorSubcoreMesh(core_axis_name="c", subcore_axis_name="s")

    @functools.partial(
        pl.kernel,
        mesh=mesh,
        out_type=jax.ShapeDtypeStruct((NC, N, D), jnp.float32),
        scratch_types=[
            pltpu.VMEM((K,), jnp.int32),
            pltpu.VMEM((K,), jnp.int32),
            pltpu.VMEM((K,), jnp.float32),
            pltpu.VMEM((K, D), jnp.float32),
            pltpu.VMEM((ZROWS, D), jnp.float32),
            pltpu.VMEM_SHARED((N, D), jnp.float32),
            pltpu.SemaphoreType.DMA,
        ],
    )
    def sc_scatter(src_hbm, dst_hbm, w_hbm, feat_hbm, out_hbm,
                   idx_s, idx_d, wbuf, rows, zbuf, agg, sem):
        cid = lax.axis_index("c")
        sid = lax.axis_index("s")
        wid = sid * NC + cid

        z16 = jnp.zeros((16,), jnp.float32)

        def zero_body(i, carry):
            for j in range(D // 16):
                zbuf[i, pl.ds(j * 16, 16)] = z16
            return carry

        lax.fori_loop(0, ZROWS, zero_body, 0)
        base_row = sid * ROWS_PER_TILE
        for r in range(ROWS_PER_TILE // ZROWS):
            pltpu.sync_copy(zbuf, agg.at[pl.ds(base_row + r * ZROWS, ZROWS)])

        @pl.when(sid == NS - 1)
        def _zero_rem():
            pltpu.sync_copy(zbuf.at[pl.ds(0, REM_ROWS)],
                            agg.at[pl.ds(NS * ROWS_PER_TILE, REM_ROWS)])

        plsc.subcore_barrier()

        ebase = wid * EW

        def chunk_body(c, carry):
            off = ebase + c * K
            pltpu.sync_copy(src_hbm.at[pl.ds(off, K)], idx_s)
            pltpu.sync_copy(dst_hbm.at[pl.ds(off, K)], idx_d)
            pltpu.sync_copy(w_hbm.at[pl.ds(off, K)], wbuf)
            pltpu.async_copy(feat_hbm.at[idx_s], rows, sem).wait()

            def scale_body(g, carry2):
                wv = wbuf[pl.ds(g * 16, 16)]
                for e in range(16):
                    w = wv[e]
                    r0 = g * 16 + e
                    for j in range(D // 16):
                        rows[r0, pl.ds(j * 16, 16)] = (
                            rows[r0, pl.ds(j * 16, 16)] * w)
                return carry2

            lax.fori_loop(0, K // 16, scale_body, 0)
            pltpu.sync_copy(rows, agg.at[idx_d], add=True)
            return carry

        lax.fori_loop(0, NCH, chunk_body, 0)

        plsc.subcore_barrier()
        pltpu.sync_copy(agg.at[pl.ds(base_row, ROWS_PER_TILE)],
                        out_hbm.at[cid, pl.ds(base_row, ROWS_PER_TILE)])

        @pl.when(sid == NS - 1)
        def _copy_rem():
            pltpu.sync_copy(
                agg.at[pl.ds(NS * ROWS_PER_TILE, REM_ROWS)],
                out_hbm.at[cid, pl.ds(NS * ROWS_PER_TILE, REM_ROWS)])

    return sc_scatter


def _tc_body(x_ref, bp_ref, k_ref, bias_ref, skip_ref, o_ref):
    kmat = k_ref[...]
    k2 = kmat * skip_ref[...]
    bsum = bp_ref[0] + bp_ref[1]
    acc = jnp.dot(x_ref[...], k2, preferred_element_type=jnp.float32,
                  precision=lax.Precision.HIGHEST)
    acc = acc + jnp.dot(bsum, kmat, preferred_element_type=jnp.float32,
                        precision=lax.Precision.HIGHEST)
    acc = acc + bias_ref[...]
    pos = acc > 0.0
    safe = jnp.where(pos, 0.0, acc)
    o_ref[...] = jnp.where(
        pos, _SELU_SCALE * acc,
        (_SELU_SCALE * _SELU_ALPHA) * (jnp.exp(safe) - 1.0))


def _tc_fused(features, bp, kmat, bias, skip):
    N, D = features.shape
    C = kmat.shape[1]
    BM = 1000
    grid = (N // BM,)
    return pl.pallas_call(
        _tc_body,
        grid=grid,
        in_specs=[
            pl.BlockSpec((BM, D), lambda i: (i, 0)),
            pl.BlockSpec((2, BM, C), lambda i: (0, i, 0)),
            pl.BlockSpec((D, C), lambda i: (0, 0)),
            pl.BlockSpec((1, C), lambda i: (0, 0)),
            pl.BlockSpec((1, C), lambda i: (0, 0)),
        ],
        out_specs=pl.BlockSpec((BM, C), lambda i: (i, 0)),
        out_shape=jax.ShapeDtypeStruct((N, C), jnp.float32),
    )(features, bp, kmat, bias, skip)


def kernel(features, edge_index, edge_weight, kernel, bias, skip_weight):
    N, D = features.shape
    C = kernel.shape[1]
    E = edge_weight.shape[0]
    dst = edge_index[0]
    src = edge_index[1]
    bp = _make_sc_scatter(N, D, E)(src, dst, edge_weight, features)
    return _tc_fused(features, bp, kernel,
                     bias.reshape(1, C), skip_weight.reshape(1, C))
